# baseline (speedup 1.0000x reference)
import functools

import jax
import jax.numpy as jnp
from jax import lax
from jax.experimental import pallas as pl
from jax.experimental.pallas import tpu as pltpu
from jax.experimental.pallas import tpu_sc as plsc

DIM = 32
NC, NS = 2, 16
NW = NC * NS
BW = 128
SEQ = 200
BBLK = 4096 // BW
UNITS = SEQ * BBLK
UPW = UNITS // NW
NSLOT = 8
NIT = UPW // NSLOT


@functools.lru_cache(maxsize=None)
def _build():
    mesh = plsc.VectorSubcoreMesh(core_axis_name="c", subcore_axis_name="s")

    def body(x_u, t_rm, out_u, idx_all, *rest):
        bufs = rest[0:NSLOT]
        gsems = rest[NSLOT:2 * NSLOT]
        osems = rest[2 * NSLOT:3 * NSLOT]
        wid = lax.axis_index("s") * NC + lax.axis_index("c")
        u0 = wid * UPW

        pltpu.sync_copy(x_u.at[pl.ds(u0, UPW)], idx_all)

        def fire_gather(u, b):
            pltpu.async_copy(t_rm.at[idx_all.at[u]], bufs[b], gsems[b])

        def drain_gather(u, b):
            pltpu.make_async_copy(
                t_rm.at[idx_all.at[u]], bufs[b], gsems[b]).wait()

        def fire_write(u, b):
            gu = u0 + u
            s = gu // BBLK
            bb = gu % BBLK
            pltpu.async_copy(
                bufs[b], out_u.at[s, pl.ds(bb * BW, BW)], osems[b])

        def drain_write(u, b):
            gu = u0 + u
            s = gu // BBLK
            bb = gu % BBLK
            pltpu.make_async_copy(
                bufs[b], out_u.at[s, pl.ds(bb * BW, BW)], osems[b]).wait()

        PF = NSLOT // 2

        for b in range(PF):
            fire_gather(b, b)

        def step_fn(it, carry):
            for b in range(NSLOT):
                u = it * NSLOT + b
                ps = (b + PF) % NSLOT

                drain_gather(u, b)
                fire_write(u, b)

                if b < PF:
                    @pl.when(it > 0)
                    def _():
                        drain_write(u - PF, ps)

                    fire_gather(u + PF, ps)
                else:
                    drain_write(u - PF, ps)

                    @pl.when(it < NIT - 1)
                    def _():
                        fire_gather(u + PF, ps)
            return carry

        lax.fori_loop(0, NIT, step_fn, 0)

        for u in range(UPW - PF, UPW):
            drain_write(u, u % NSLOT)

    return pl.kernel(
        body,
        out_type=jax.ShapeDtypeStruct((SEQ, 4096, DIM), jnp.float32),
        mesh=mesh,
        scratch_types=(
            [pltpu.VMEM((UPW, BW), jnp.int32)]
            + [pltpu.VMEM((BW, DIM), jnp.float32) for _ in range(NSLOT)]
            + [pltpu.SemaphoreType.DMA for _ in range(2 * NSLOT)]
        ),
        compiler_params=pltpu.CompilerParams(
            use_tc_tiling_on_sc=False, needs_layout_passes=False),
    )


@jax.jit
def kernel(x, table):
    b, s = x.shape
    x_u = (jnp.transpose(x).reshape(UNITS, BW) * 4).astype(jnp.int32)
    tp = jax.lax.optimization_barrier(jnp.pad(table, ((0, 0), (0, 96))))
    t_rm = tp.reshape(4000000, DIM)
    out_u = _build()(x_u, t_rm)
    return jnp.transpose(out_u, (1, 0, 2))


# speedup vs baseline: 1.3512x; 1.3512x over previous
import functools

import jax
import jax.numpy as jnp
from jax import lax
from jax.experimental import pallas as pl
from jax.experimental.pallas import tpu as pltpu
from jax.experimental.pallas import tpu_sc as plsc

DIM = 32
NC, NS = 2, 16
NW = NC * NS
BW = 128
SEQ = 200
BBLK = 4096 // BW
UNITS = SEQ * BBLK
UPW = UNITS // NW
NSLOT = 8
NIT = UPW // NSLOT


@functools.lru_cache(maxsize=None)
def _build():
    mesh = plsc.VectorSubcoreMesh(core_axis_name="c", subcore_axis_name="s")

    def body(x_u, t_rm, out_u, idx_all, *rest):
        bufs = rest[0:NSLOT]
        gsems = rest[NSLOT:2 * NSLOT]
        osems = rest[2 * NSLOT:3 * NSLOT]
        wid = lax.axis_index("s") * NC + lax.axis_index("c")
        u0 = wid * UPW

        pltpu.sync_copy(x_u.at[pl.ds(u0, UPW)], idx_all)

        def fire_gather(u, b):
            pltpu.async_copy(t_rm.at[idx_all.at[u]], bufs[b], gsems[b])

        def drain_gather(u, b):
            pltpu.make_async_copy(
                t_rm.at[idx_all.at[u]], bufs[b], gsems[b]).wait()

        def fire_write(u, b):
            gu = u0 + u
            s = gu // BBLK
            bb = gu % BBLK
            pltpu.async_copy(
                bufs[b], out_u.at[s, pl.ds(bb * BW, BW)], osems[b])

        def drain_write(u, b):
            gu = u0 + u
            s = gu // BBLK
            bb = gu % BBLK
            pltpu.make_async_copy(
                bufs[b], out_u.at[s, pl.ds(bb * BW, BW)], osems[b]).wait()

        PF = NSLOT // 2

        for b in range(PF):
            fire_gather(b, b)

        def step_fn(it, carry):
            for b in range(NSLOT):
                u = it * NSLOT + b
                ps = (b + PF) % NSLOT

                drain_gather(u, b)
                fire_write(u, b)

                if b < PF:
                    @pl.when(it > 0)
                    def _():
                        drain_write(u - PF, ps)

                    fire_gather(u + PF, ps)
                else:
                    drain_write(u - PF, ps)

                    @pl.when(it < NIT - 1)
                    def _():
                        fire_gather(u + PF, ps)
            return carry

        lax.fori_loop(0, NIT, step_fn, 0)

        for u in range(UPW - PF, UPW):
            drain_write(u, u % NSLOT)

    return pl.kernel(
        body,
        out_type=jax.ShapeDtypeStruct((SEQ, 4096, DIM), jnp.float32),
        mesh=mesh,
        scratch_types=(
            [pltpu.VMEM((UPW, BW), jnp.int32)]
            + [pltpu.VMEM((BW, DIM), jnp.float32) for _ in range(NSLOT)]
            + [pltpu.SemaphoreType.DMA for _ in range(2 * NSLOT)]
        ),
        compiler_params=pltpu.CompilerParams(
            use_tc_tiling_on_sc=False, needs_layout_passes=False),
    )


TCC = 4096


@functools.lru_cache(maxsize=None)
def _tc_pad_transpose():
    grid = (1000000 + TCC - 1) // TCC

    def k(tt_ref, out_ref):
        out_ref[:, pl.ds(0, DIM)] = jnp.transpose(tt_ref[...], (1, 0))

    return pl.pallas_call(
        k,
        grid=(grid,),
        in_specs=[pl.BlockSpec((DIM, TCC), lambda g: (0, g))],
        out_specs=pl.BlockSpec((TCC, 128), lambda g: (g, 0)),
        out_shape=jax.ShapeDtypeStruct((1000000, 128), jnp.float32),
    )


@jax.jit
def kernel(x, table):
    b, s = x.shape
    x_u = (jnp.transpose(x).reshape(UNITS, BW) * 4).astype(jnp.int32)
    tp = _tc_pad_transpose()(jnp.transpose(table))
    t_rm = tp.reshape(4000000, DIM)
    out_u = _build()(x_u, t_rm)
    return jnp.transpose(out_u, (1, 0, 2))


# speedup vs baseline: 1.4858x; 1.0996x over previous
import functools

import jax
import jax.numpy as jnp
from jax import lax
from jax.experimental import pallas as pl
from jax.experimental.pallas import tpu as pltpu
from jax.experimental.pallas import tpu_sc as plsc

DIM = 32
NC, NS = 2, 16
NW = NC * NS
BW = 128
SEQ = 200
BBLK = 4096 // BW
UNITS = SEQ * BBLK
UPW = UNITS // NW
NSLOT = 8
NIT = UPW // NSLOT


@functools.lru_cache(maxsize=None)
def _build():
    mesh = plsc.VectorSubcoreMesh(core_axis_name="c", subcore_axis_name="s")

    def body(x_u, t_rm, out_u, idx_all, *rest):
        bufs = rest[0:NSLOT]
        gsems = rest[NSLOT:2 * NSLOT]
        osems = rest[2 * NSLOT:3 * NSLOT]
        wid = lax.axis_index("s") * NC + lax.axis_index("c")
        u0 = wid * UPW

        pltpu.sync_copy(x_u.at[pl.ds(u0, UPW)], idx_all)

        def fire_gather(u, b):
            pltpu.async_copy(t_rm.at[idx_all.at[u]], bufs[b], gsems[b])

        def drain_gather(u, b):
            pltpu.make_async_copy(
                t_rm.at[idx_all.at[u]], bufs[b], gsems[b]).wait()

        def fire_write(u, b):
            gu = u0 + u
            s = gu // BBLK
            bb = gu % BBLK
            pltpu.async_copy(
                bufs[b], out_u.at[s, pl.ds(bb * BW, BW)], osems[b])

        def drain_write(u, b):
            gu = u0 + u
            s = gu // BBLK
            bb = gu % BBLK
            pltpu.make_async_copy(
                bufs[b], out_u.at[s, pl.ds(bb * BW, BW)], osems[b]).wait()

        PF = NSLOT // 2

        for b in range(PF):
            fire_gather(b, b)

        def step_fn(it, carry):
            for b in range(NSLOT):
                u = it * NSLOT + b
                ps = (b + PF) % NSLOT

                drain_gather(u, b)
                fire_write(u, b)

                if b < PF:
                    @pl.when(it > 0)
                    def _():
                        drain_write(u - PF, ps)

                    fire_gather(u + PF, ps)
                else:
                    drain_write(u - PF, ps)

                    @pl.when(it < NIT - 1)
                    def _():
                        fire_gather(u + PF, ps)
            return carry

        lax.fori_loop(0, NIT, step_fn, 0)

        for u in range(UPW - PF, UPW):
            drain_write(u, u % NSLOT)

    return pl.kernel(
        body,
        out_type=jax.ShapeDtypeStruct((SEQ, 4096, DIM), jnp.float32),
        mesh=mesh,
        scratch_types=(
            [pltpu.VMEM((UPW, BW), jnp.int32)]
            + [pltpu.VMEM((BW, DIM), jnp.float32) for _ in range(NSLOT)]
            + [pltpu.SemaphoreType.DMA for _ in range(2 * NSLOT)]
        ),
        compiler_params=pltpu.CompilerParams(
            use_tc_tiling_on_sc=False, needs_layout_passes=False),
    )


TCC = 8192


@functools.lru_cache(maxsize=None)
def _tc_pad_transpose():
    grid = (1000000 + TCC - 1) // TCC

    def k(tt_ref, out_ref):
        out_ref[:, pl.ds(0, DIM)] = jnp.transpose(tt_ref[...], (1, 0))

    return pl.pallas_call(
        k,
        grid=(grid,),
        in_specs=[pl.BlockSpec((DIM, TCC), lambda g: (0, g))],
        out_specs=pl.BlockSpec((TCC, 128), lambda g: (g, 0)),
        out_shape=jax.ShapeDtypeStruct((1000000, 128), jnp.float32),
    )


@jax.jit
def kernel(x, table):
    b, s = x.shape
    x_u = (jnp.transpose(x).reshape(UNITS, BW) * 4).astype(jnp.int32)
    tp = _tc_pad_transpose()(jnp.transpose(table))
    t_rm = tp.reshape(4000000, DIM)
    out_u = _build()(x_u, t_rm)
    return jnp.transpose(out_u, (1, 0, 2))
